# baseline (speedup 1.0000x reference)
import functools

import jax
import jax.numpy as jnp
from jax import lax
from jax.experimental import pallas as pl
from jax.experimental.pallas import tpu as pltpu
from jax.experimental.pallas import tpu_sc as plsc

N = 16777216
K = 16
WMAX = 4.0
EPS = 1e-06
L = 16
NC, NS = 2, 16
NW = NC * NS
PER_W = N // NW
CHUNK = 16384
NROT_H = 16
NCH = PER_W // CHUNK

_MESH = plsc.VectorSubcoreMesh(core_axis_name="c", subcore_axis_name="s")


def _wid():
    return lax.axis_index("c") * NS + lax.axis_index("s")


def _splat(scal_ref, i):
    return plsc.load_gather(scal_ref, [jnp.full((L,), i, jnp.int32)])


def _start_in(values, base, ci, buf, sem):
    pltpu.make_async_copy(values.at[pl.ds(base + ci * CHUNK, CHUNK)], buf,
                          sem).start()


def _wait_in(values, buf, sem):
    pltpu.make_async_copy(values.at[pl.ds(0, CHUNK)], buf, sem).wait()


def _bint(v, p, q):
    return (v * p + q).astype(jnp.int32)


def _stats_body(values, parts, buf0, buf1, stage, sem0, sem1):
    wid = _wid()
    base = wid * PER_W
    _start_in(values, base, 0, buf0, sem0)
    _start_in(values, base, 1, buf1, sem1)

    neg = jnp.full((L,), -3.4e38, jnp.float32)
    NACC = 4
    init = tuple(jnp.zeros((L,), jnp.float32) for _ in range(2 * NACC)) + \
        tuple(neg for _ in range(NACC)) + tuple(-neg for _ in range(NACC))

    def chunk_pair(i, carry):
        for sub, buf, sem in ((0, buf0, sem0), (1, buf1, sem1)):
            ci = 2 * i + sub
            _wait_in(values, buf, sem)

            @plsc.parallel_loop(0, CHUNK // L, NACC, unroll=4, carry=carry)
            def inner(j, c):
                c = list(c)
                for kk in range(NACC):
                    v = buf[pl.ds((j + kk) * L, L)]
                    c[kk] = c[kk] + v
                    c[NACC + kk] = c[NACC + kk] + v * v
                    c[2 * NACC + kk] = jnp.maximum(c[2 * NACC + kk], v)
                    c[3 * NACC + kk] = jnp.minimum(c[3 * NACC + kk], v)
                return tuple(c)

            carry = inner

            @pl.when(ci + 2 < NCH)
            def _():
                _start_in(values, base, ci + 2, buf, sem)
        return carry

    fin = lax.fori_loop(0, NCH // 2, chunk_pair, init)
    s = fin[0] + fin[1] + fin[2] + fin[3]
    sq = fin[4] + fin[5] + fin[6] + fin[7]
    mx = jnp.maximum(jnp.maximum(fin[8], fin[9]),
                     jnp.maximum(fin[10], fin[11]))
    mn = jnp.minimum(jnp.minimum(fin[12], fin[13]),
                     jnp.minimum(fin[14], fin[15]))
    stage[0] = s
    stage[1] = sq
    stage[2] = mx
    stage[3] = mn
    pltpu.sync_copy(stage, parts.at[wid])


_stats = pl.kernel(
    _stats_body,
    out_type=jax.ShapeDtypeStruct((NW, 4, L), jnp.float32),
    mesh=_MESH,
    compiler_params=pltpu.CompilerParams(needs_layout_passes=False),
    scratch_types=[
        pltpu.VMEM((CHUNK,), jnp.float32),
        pltpu.VMEM((CHUNK,), jnp.float32),
        pltpu.VMEM((4, L), jnp.float32),
        pltpu.SemaphoreType.DMA,
        pltpu.SemaphoreType.DMA,
    ],
)


def _hist_body(values, scal, hists, buf0, buf1, scal_v, hist_v, hist8_v,
               sem0, sem1):
    wid = _wid()
    base = wid * PER_W
    pltpu.sync_copy(scal, scal_v)
    _start_in(values, base, 0, buf0, sem0)
    _start_in(values, base, 1, buf1, sem1)

    p = _splat(scal_v, 5)
    q = _splat(scal_v, 6)
    ones = jnp.ones((L,), jnp.float32)
    NROT = NROT_H
    for r in range(NROT * L):
        hist8_v[pl.ds(r * L, L)] = jnp.zeros((L,), jnp.float32)
    lane16 = jax.lax.iota(jnp.int32, L) * L

    def chunk_pair(i, carry):
        for sub, buf, sem in ((0, buf0, sem0), (1, buf1, sem1)):
            ci = 2 * i + sub
            _wait_in(values, buf, sem)

            @plsc.parallel_loop(0, CHUNK // L, 1, unroll=16)
            def _(j):
                v = buf[pl.ds(j * L, L)]
                b = jnp.clip(_bint(v, p, q), 0, K - 1)
                off = lane16 + (j & (NROT - 1)) * (L * L)
                plsc.addupdate_scatter(hist8_v, [b + off], ones)

            @pl.when(ci + 2 < NCH)
            def _():
                _start_in(values, base, ci + 2, buf, sem)
        return carry

    lax.fori_loop(0, NCH // 2, chunk_pair, 0)
    acc = jnp.zeros((L,), jnp.float32)
    for r in range(NROT * L):
        acc = acc + hist8_v[pl.ds(r * L, L)]
    hist_v[...] = acc
    pltpu.sync_copy(hist_v, hists.at[wid])


_hist = pl.kernel(
    _hist_body,
    out_type=jax.ShapeDtypeStruct((NW, L), jnp.float32),
    mesh=_MESH,
    compiler_params=pltpu.CompilerParams(needs_layout_passes=False),
    scratch_types=[
        pltpu.VMEM((CHUNK,), jnp.float32),
        pltpu.VMEM((CHUNK,), jnp.float32),
        pltpu.VMEM((L,), jnp.float32),
        pltpu.VMEM((L,), jnp.float32),
        pltpu.VMEM((NROT_H * L * L,), jnp.float32),
        pltpu.SemaphoreType.DMA,
        pltpu.SemaphoreType.DMA,
    ],
)


def _apply_body(values, scal, wd, out, tail32,
                buf0, buf1, obuf0, obuf1, tbuf0, tbuf1,
                scal_v, wd_v,
                sem0, sem1, osem0, osem1, tsem0, tsem1):
    wid = _wid()
    base = wid * PER_W
    pltpu.sync_copy(scal, scal_v)
    pltpu.sync_copy(wd, wd_v)
    _start_in(values, base, 0, buf0, sem0)
    _start_in(values, base, 1, buf1, sem1)

    isd = _splat(scal_v, 2)
    nmi = _splat(scal_v, 4)
    p = _splat(scal_v, 5)
    q = _splat(scal_v, 6)

    def chunk_pair(i, carry):
        for sub, buf, sem, obuf, osem, tbuf, tsem in (
                (0, buf0, sem0, obuf0, osem0, tbuf0, tsem0),
                (1, buf1, sem1, obuf1, osem1, tbuf1, tsem1)):
            ci = 2 * i + sub
            _wait_in(values, buf, sem)

            @pl.when(ci >= 2)
            def _():
                pltpu.make_async_copy(obuf, out.at[pl.ds(0, CHUNK)],
                                      osem).wait()
                pltpu.make_async_copy(tbuf, tail32.at[pl.ds(0, CHUNK)],
                                      tsem).wait()

            ione, izero = carry

            @plsc.parallel_loop(0, CHUNK // L, 1, unroll=16)
            def _(j):
                v = buf[pl.ds(j * L, L)]
                t = v * p + q
                b = jnp.clip(t.astype(jnp.int32), 0, K - 1)
                w = plsc.load_gather(wd_v, [b])
                z = v * isd + nmi
                obuf[pl.ds(j * L, L)] = w * z
                tbuf[pl.ds(j * L, L)] = jnp.where(
                    (t < 1.0) | (t >= float(K - 1)), ione, izero)

            pltpu.make_async_copy(obuf, out.at[pl.ds(base + ci * CHUNK,
                                                     CHUNK)], osem).start()
            pltpu.make_async_copy(
                tbuf, tail32.at[pl.ds(base + ci * CHUNK, CHUNK)],
                tsem).start()

            @pl.when(ci + 2 < NCH)
            def _():
                _start_in(values, base, ci + 2, buf, sem)
        return carry

    lax.fori_loop(0, NCH // 2, chunk_pair,
                  (jnp.ones((L,), jnp.int32), jnp.zeros((L,), jnp.int32)))
    for obuf, osem, tbuf, tsem in ((obuf0, osem0, tbuf0, tsem0),
                                   (obuf1, osem1, tbuf1, tsem1)):
        pltpu.make_async_copy(obuf, out.at[pl.ds(0, CHUNK)], osem).wait()
        pltpu.make_async_copy(tbuf, tail32.at[pl.ds(0, CHUNK)],
                              tsem).wait()


_apply = pl.kernel(
    _apply_body,
    out_type=(
        jax.ShapeDtypeStruct((N,), jnp.float32),
        jax.ShapeDtypeStruct((N,), jnp.int32),
    ),
    mesh=_MESH,
    compiler_params=pltpu.CompilerParams(needs_layout_passes=False),
    scratch_types=[
        pltpu.VMEM((CHUNK,), jnp.float32),
        pltpu.VMEM((CHUNK,), jnp.float32),
        pltpu.VMEM((CHUNK,), jnp.float32),
        pltpu.VMEM((CHUNK,), jnp.float32),
        pltpu.VMEM((CHUNK,), jnp.int32),
        pltpu.VMEM((CHUNK,), jnp.int32),
        pltpu.VMEM((L,), jnp.float32),
        pltpu.VMEM((L,), jnp.float32),
        pltpu.SemaphoreType.DMA,
        pltpu.SemaphoreType.DMA,
        pltpu.SemaphoreType.DMA,
        pltpu.SemaphoreType.DMA,
        pltpu.SemaphoreType.DMA,
        pltpu.SemaphoreType.DMA,
    ],
)


def kernel(values, k):
    parts = _stats(values)
    n = jnp.float32(N)
    s = jnp.sum(parts[:, 0, :])
    sq = jnp.sum(parts[:, 1, :])
    vmax = jnp.max(parts[:, 2, :])
    vmin = jnp.min(parts[:, 3, :])
    mu = s / n
    var = sq / n - mu * mu
    sd = jnp.sqrt(jnp.clip(var, EPS))
    zmax = jnp.clip(jnp.maximum(jnp.abs(vmax - mu), jnp.abs(vmin - mu)) / sd,
                    3.0, 8.0)
    inv_h = (K / 2) / zmax
    scal = jnp.zeros((L,), jnp.float32)
    isd = 1.0 / sd
    scal = scal.at[1].set(mu).at[2].set(isd).at[3].set(zmax)
    scal = scal.at[4].set(-mu * isd)
    scal = scal.at[5].set(isd * inv_h)
    scal = scal.at[6].set((zmax - mu * isd) * inv_h)


    hists = _hist(values, scal)
    c = jnp.sum(hists, axis=0)
    pos = c > 0
    c_mean = jnp.where(jnp.any(pos),
                       jnp.sum(jnp.where(pos, c, 0.0)) /
                       jnp.maximum(jnp.sum(pos.astype(jnp.float32)), 1.0),
                       jnp.float32(1.0))
    wd_bins = jnp.clip(c_mean / (c + EPS), 1.0, WMAX)
    out, tail32 = _apply(values, scal, wd_bins)
    tail = tail32 != 0
    return out, c, wd_bins, tail


# speedup vs baseline: 1.2164x; 1.2164x over previous
import functools

import jax
import jax.numpy as jnp
from jax import lax
from jax.experimental import pallas as pl
from jax.experimental.pallas import tpu as pltpu
from jax.experimental.pallas import tpu_sc as plsc

N = 16777216
K = 16
WMAX = 4.0
EPS = 1e-06
L = 16
NC, NS = 2, 16
NW = NC * NS
PER_W = N // NW
CHUNK = 16384
NROT_H = 8
NCH = PER_W // CHUNK

_MESH = plsc.VectorSubcoreMesh(core_axis_name="c", subcore_axis_name="s")


def _wid():
    return lax.axis_index("c") * NS + lax.axis_index("s")


def _splat(scal_ref, i):
    return plsc.load_gather(scal_ref, [jnp.full((L,), i, jnp.int32)])


def _start_in(values, base, ci, buf, sem):
    pltpu.make_async_copy(values.at[pl.ds(base + ci * CHUNK, CHUNK)], buf,
                          sem).start()


def _wait_in(values, buf, sem):
    pltpu.make_async_copy(values.at[pl.ds(0, CHUNK)], buf, sem).wait()


def _bint(v, p, q):
    return (v * p + q).astype(jnp.int32)


def _stats_body(values, parts, buf0, buf1, stage, sem0, sem1):
    wid = _wid()
    base = wid * PER_W
    _start_in(values, base, 0, buf0, sem0)
    _start_in(values, base, 1, buf1, sem1)

    neg = jnp.full((L,), -3.4e38, jnp.float32)
    NACC = 4
    init = tuple(jnp.zeros((L,), jnp.float32) for _ in range(2 * NACC)) + \
        tuple(neg for _ in range(NACC)) + tuple(-neg for _ in range(NACC))

    def chunk_pair(i, carry):
        for sub, buf, sem in ((0, buf0, sem0), (1, buf1, sem1)):
            ci = 2 * i + sub
            _wait_in(values, buf, sem)

            @plsc.parallel_loop(0, CHUNK // L, NACC, unroll=4, carry=carry)
            def inner(j, c):
                c = list(c)
                for kk in range(NACC):
                    v = buf[pl.ds((j + kk) * L, L)]
                    c[kk] = c[kk] + v
                    c[NACC + kk] = c[NACC + kk] + v * v
                    c[2 * NACC + kk] = jnp.maximum(c[2 * NACC + kk], v)
                    c[3 * NACC + kk] = jnp.minimum(c[3 * NACC + kk], v)
                return tuple(c)

            carry = inner

            @pl.when(ci + 2 < NCH)
            def _():
                _start_in(values, base, ci + 2, buf, sem)
        return carry

    fin = lax.fori_loop(0, NCH // 2, chunk_pair, init)
    s = fin[0] + fin[1] + fin[2] + fin[3]
    sq = fin[4] + fin[5] + fin[6] + fin[7]
    mx = jnp.maximum(jnp.maximum(fin[8], fin[9]),
                     jnp.maximum(fin[10], fin[11]))
    mn = jnp.minimum(jnp.minimum(fin[12], fin[13]),
                     jnp.minimum(fin[14], fin[15]))
    stage[0] = s
    stage[1] = sq
    stage[2] = mx
    stage[3] = mn
    pltpu.sync_copy(stage, parts.at[wid])


_stats = pl.kernel(
    _stats_body,
    out_type=jax.ShapeDtypeStruct((NW, 4, L), jnp.float32),
    mesh=_MESH,
    compiler_params=pltpu.CompilerParams(needs_layout_passes=False),
    scratch_types=[
        pltpu.VMEM((CHUNK,), jnp.float32),
        pltpu.VMEM((CHUNK,), jnp.float32),
        pltpu.VMEM((4, L), jnp.float32),
        pltpu.SemaphoreType.DMA,
        pltpu.SemaphoreType.DMA,
    ],
)


def _hist_body(values, scal, hists, buf0, buf1, scal_v, hist_v, hist8_v,
               sem0, sem1):
    wid = _wid()
    base = wid * PER_W
    pltpu.sync_copy(scal, scal_v)
    _start_in(values, base, 0, buf0, sem0)
    _start_in(values, base, 1, buf1, sem1)

    p = _splat(scal_v, 5)
    q = _splat(scal_v, 6)
    ones = jnp.ones((L,), jnp.float32)
    NROT = NROT_H
    for r in range(NROT * L):
        hist8_v[pl.ds(r * L, L)] = jnp.zeros((L,), jnp.float32)
    lane16 = jax.lax.iota(jnp.int32, L) * L

    def chunk_pair(i, carry):
        for sub, buf, sem in ((0, buf0, sem0), (1, buf1, sem1)):
            ci = 2 * i + sub
            _wait_in(values, buf, sem)

            @plsc.parallel_loop(0, CHUNK // L, 1, unroll=8)
            def _(j):
                v = buf[pl.ds(j * L, L)]
                b = jnp.clip(_bint(v, p, q), 0, K - 1)
                off = lane16 + (j & (NROT - 1)) * (L * L)
                plsc.addupdate_scatter(hist8_v, [b + off], ones)

            @pl.when(ci + 2 < NCH)
            def _():
                _start_in(values, base, ci + 2, buf, sem)
        return carry

    lax.fori_loop(0, NCH // 2, chunk_pair, 0)
    acc = jnp.zeros((L,), jnp.float32)
    for r in range(NROT * L):
        acc = acc + hist8_v[pl.ds(r * L, L)]
    hist_v[...] = acc
    pltpu.sync_copy(hist_v, hists.at[wid])


_hist = pl.kernel(
    _hist_body,
    out_type=jax.ShapeDtypeStruct((NW, L), jnp.float32),
    mesh=_MESH,
    compiler_params=pltpu.CompilerParams(needs_layout_passes=False),
    scratch_types=[
        pltpu.VMEM((CHUNK,), jnp.float32),
        pltpu.VMEM((CHUNK,), jnp.float32),
        pltpu.VMEM((L,), jnp.float32),
        pltpu.VMEM((L,), jnp.float32),
        pltpu.VMEM((NROT_H * L * L,), jnp.float32),
        pltpu.SemaphoreType.DMA,
        pltpu.SemaphoreType.DMA,
    ],
)


def _apply_body(values, scal, wd, out, tail32,
                buf0, buf1, obuf0, obuf1, tbuf0, tbuf1,
                scal_v, wd_v,
                sem0, sem1, osem0, osem1, tsem0, tsem1):
    wid = _wid()
    base = wid * PER_W
    pltpu.sync_copy(scal, scal_v)
    pltpu.sync_copy(wd, wd_v)
    _start_in(values, base, 0, buf0, sem0)
    _start_in(values, base, 1, buf1, sem1)

    isd = _splat(scal_v, 2)
    nmi = _splat(scal_v, 4)
    p = _splat(scal_v, 5)
    q = _splat(scal_v, 6)

    def chunk_pair(i, carry):
        for sub, buf, sem, obuf, osem, tbuf, tsem in (
                (0, buf0, sem0, obuf0, osem0, tbuf0, tsem0),
                (1, buf1, sem1, obuf1, osem1, tbuf1, tsem1)):
            ci = 2 * i + sub
            _wait_in(values, buf, sem)

            @pl.when(ci >= 2)
            def _():
                pltpu.make_async_copy(obuf, out.at[pl.ds(0, CHUNK)],
                                      osem).wait()
                pltpu.make_async_copy(tbuf, tail32.at[pl.ds(0, CHUNK)],
                                      tsem).wait()

            ione, izero = carry

            @plsc.parallel_loop(0, CHUNK // L, 1, unroll=8)
            def _(j):
                v = buf[pl.ds(j * L, L)]
                t = v * p + q
                b = jnp.clip(t.astype(jnp.int32), 0, K - 1)
                w = plsc.load_gather(wd_v, [b])
                z = v * isd + nmi
                obuf[pl.ds(j * L, L)] = w * z
                tbuf[pl.ds(j * L, L)] = jnp.where(
                    (t < 1.0) | (t >= float(K - 1)), ione, izero)

            pltpu.make_async_copy(obuf, out.at[pl.ds(base + ci * CHUNK,
                                                     CHUNK)], osem).start()
            pltpu.make_async_copy(
                tbuf, tail32.at[pl.ds(base + ci * CHUNK, CHUNK)],
                tsem).start()

            @pl.when(ci + 2 < NCH)
            def _():
                _start_in(values, base, ci + 2, buf, sem)
        return carry

    lax.fori_loop(0, NCH // 2, chunk_pair,
                  (jnp.ones((L,), jnp.int32), jnp.zeros((L,), jnp.int32)))
    for obuf, osem, tbuf, tsem in ((obuf0, osem0, tbuf0, tsem0),
                                   (obuf1, osem1, tbuf1, tsem1)):
        pltpu.make_async_copy(obuf, out.at[pl.ds(0, CHUNK)], osem).wait()
        pltpu.make_async_copy(tbuf, tail32.at[pl.ds(0, CHUNK)],
                              tsem).wait()


_apply = pl.kernel(
    _apply_body,
    out_type=(
        jax.ShapeDtypeStruct((N,), jnp.float32),
        jax.ShapeDtypeStruct((N,), jnp.int32),
    ),
    mesh=_MESH,
    compiler_params=pltpu.CompilerParams(needs_layout_passes=False),
    scratch_types=[
        pltpu.VMEM((CHUNK,), jnp.float32),
        pltpu.VMEM((CHUNK,), jnp.float32),
        pltpu.VMEM((CHUNK,), jnp.float32),
        pltpu.VMEM((CHUNK,), jnp.float32),
        pltpu.VMEM((CHUNK,), jnp.int32),
        pltpu.VMEM((CHUNK,), jnp.int32),
        pltpu.VMEM((L,), jnp.float32),
        pltpu.VMEM((L,), jnp.float32),
        pltpu.SemaphoreType.DMA,
        pltpu.SemaphoreType.DMA,
        pltpu.SemaphoreType.DMA,
        pltpu.SemaphoreType.DMA,
        pltpu.SemaphoreType.DMA,
        pltpu.SemaphoreType.DMA,
    ],
)


def kernel(values, k):
    parts = _stats(values)
    n = jnp.float32(N)
    s = jnp.sum(parts[:, 0, :])
    sq = jnp.sum(parts[:, 1, :])
    vmax = jnp.max(parts[:, 2, :])
    vmin = jnp.min(parts[:, 3, :])
    mu = s / n
    var = sq / n - mu * mu
    sd = jnp.sqrt(jnp.clip(var, EPS))
    zmax = jnp.clip(jnp.maximum(jnp.abs(vmax - mu), jnp.abs(vmin - mu)) / sd,
                    3.0, 8.0)
    inv_h = (K / 2) / zmax
    scal = jnp.zeros((L,), jnp.float32)
    isd = 1.0 / sd
    scal = scal.at[1].set(mu).at[2].set(isd).at[3].set(zmax)
    scal = scal.at[4].set(-mu * isd)
    scal = scal.at[5].set(isd * inv_h)
    scal = scal.at[6].set((zmax - mu * isd) * inv_h)


    hists = _hist(values, scal)
    c = jnp.sum(hists, axis=0)
    pos = c > 0
    c_mean = jnp.where(jnp.any(pos),
                       jnp.sum(jnp.where(pos, c, 0.0)) /
                       jnp.maximum(jnp.sum(pos.astype(jnp.float32)), 1.0),
                       jnp.float32(1.0))
    wd_bins = jnp.clip(c_mean / (c + EPS), 1.0, WMAX)
    out, tail32 = _apply(values, scal, wd_bins)
    tail = tail32 != 0
    return out, c, wd_bins, tail


# speedup vs baseline: 1.2245x; 1.0067x over previous
import functools

import jax
import jax.numpy as jnp
from jax import lax
from jax.experimental import pallas as pl
from jax.experimental.pallas import tpu as pltpu
from jax.experimental.pallas import tpu_sc as plsc

N = 16777216
K = 16
WMAX = 4.0
EPS = 1e-06
L = 16
NC, NS = 2, 16
NW = NC * NS
PER_W = N // NW
CHUNK = 16384
CHUNK_A = 32768
NCH_A = PER_W // CHUNK_A
NROT_H = 8
NCH = PER_W // CHUNK

_MESH = plsc.VectorSubcoreMesh(core_axis_name="c", subcore_axis_name="s")


def _wid():
    return lax.axis_index("c") * NS + lax.axis_index("s")


def _splat(scal_ref, i):
    return plsc.load_gather(scal_ref, [jnp.full((L,), i, jnp.int32)])


def _start_in(values, base, ci, buf, sem, size=CHUNK):
    pltpu.make_async_copy(values.at[pl.ds(base + ci * size, size)], buf,
                          sem).start()


def _wait_in(values, buf, sem, size=CHUNK):
    pltpu.make_async_copy(values.at[pl.ds(0, size)], buf, sem).wait()


def _bint(v, p, q):
    return (v * p + q).astype(jnp.int32)


def _stats_body(values, parts, buf0, buf1, stage, sem0, sem1):
    wid = _wid()
    base = wid * PER_W
    _start_in(values, base, 0, buf0, sem0, CHUNK_A)
    _start_in(values, base, 1, buf1, sem1, CHUNK_A)

    neg = jnp.full((L,), -3.4e38, jnp.float32)
    NACC = 4
    init = tuple(jnp.zeros((L,), jnp.float32) for _ in range(2 * NACC)) + \
        tuple(neg for _ in range(NACC)) + tuple(-neg for _ in range(NACC))

    def chunk_pair(i, carry):
        for sub, buf, sem in ((0, buf0, sem0), (1, buf1, sem1)):
            ci = 2 * i + sub
            _wait_in(values, buf, sem, CHUNK_A)

            @plsc.parallel_loop(0, CHUNK_A // L, NACC, unroll=4, carry=carry)
            def inner(j, c):
                c = list(c)
                for kk in range(NACC):
                    v = buf[pl.ds((j + kk) * L, L)]
                    c[kk] = c[kk] + v
                    c[NACC + kk] = c[NACC + kk] + v * v
                    c[2 * NACC + kk] = jnp.maximum(c[2 * NACC + kk], v)
                    c[3 * NACC + kk] = jnp.minimum(c[3 * NACC + kk], v)
                return tuple(c)

            carry = inner

            @pl.when(ci + 2 < NCH_A)
            def _():
                _start_in(values, base, ci + 2, buf, sem, CHUNK_A)
        return carry

    fin = lax.fori_loop(0, NCH_A // 2, chunk_pair, init)
    s = fin[0] + fin[1] + fin[2] + fin[3]
    sq = fin[4] + fin[5] + fin[6] + fin[7]
    mx = jnp.maximum(jnp.maximum(fin[8], fin[9]),
                     jnp.maximum(fin[10], fin[11]))
    mn = jnp.minimum(jnp.minimum(fin[12], fin[13]),
                     jnp.minimum(fin[14], fin[15]))
    stage[0] = s
    stage[1] = sq
    stage[2] = mx
    stage[3] = mn
    pltpu.sync_copy(stage, parts.at[wid])


_stats = pl.kernel(
    _stats_body,
    out_type=jax.ShapeDtypeStruct((NW, 4, L), jnp.float32),
    mesh=_MESH,
    compiler_params=pltpu.CompilerParams(needs_layout_passes=False),
    scratch_types=[
        pltpu.VMEM((CHUNK_A,), jnp.float32),
        pltpu.VMEM((CHUNK_A,), jnp.float32),
        pltpu.VMEM((4, L), jnp.float32),
        pltpu.SemaphoreType.DMA,
        pltpu.SemaphoreType.DMA,
    ],
)


def _hist_body(values, scal, hists, buf0, buf1, scal_v, hist_v, hist8_v,
               sem0, sem1):
    wid = _wid()
    base = wid * PER_W
    pltpu.sync_copy(scal, scal_v)
    _start_in(values, base, 0, buf0, sem0, CHUNK_A)
    _start_in(values, base, 1, buf1, sem1, CHUNK_A)

    p = _splat(scal_v, 5)
    q = _splat(scal_v, 6)
    ones = jnp.ones((L,), jnp.float32)
    NROT = NROT_H
    for r in range(NROT * L):
        hist8_v[pl.ds(r * L, L)] = jnp.zeros((L,), jnp.float32)
    lane16 = jax.lax.iota(jnp.int32, L) * L

    def chunk_pair(i, carry):
        for sub, buf, sem in ((0, buf0, sem0), (1, buf1, sem1)):
            ci = 2 * i + sub
            _wait_in(values, buf, sem, CHUNK_A)

            @plsc.parallel_loop(0, CHUNK_A // L, 1, unroll=8)
            def _(j):
                v = buf[pl.ds(j * L, L)]
                b = jnp.clip(_bint(v, p, q), 0, K - 1)
                off = lane16 + (j & (NROT - 1)) * (L * L)
                plsc.addupdate_scatter(hist8_v, [b + off], ones)

            @pl.when(ci + 2 < NCH_A)
            def _():
                _start_in(values, base, ci + 2, buf, sem, CHUNK_A)
        return carry

    lax.fori_loop(0, NCH_A // 2, chunk_pair, 0)
    acc = jnp.zeros((L,), jnp.float32)
    for r in range(NROT * L):
        acc = acc + hist8_v[pl.ds(r * L, L)]
    hist_v[...] = acc
    pltpu.sync_copy(hist_v, hists.at[wid])


_hist = pl.kernel(
    _hist_body,
    out_type=jax.ShapeDtypeStruct((NW, L), jnp.float32),
    mesh=_MESH,
    compiler_params=pltpu.CompilerParams(needs_layout_passes=False),
    scratch_types=[
        pltpu.VMEM((CHUNK_A,), jnp.float32),
        pltpu.VMEM((CHUNK_A,), jnp.float32),
        pltpu.VMEM((L,), jnp.float32),
        pltpu.VMEM((L,), jnp.float32),
        pltpu.VMEM((NROT_H * L * L,), jnp.float32),
        pltpu.SemaphoreType.DMA,
        pltpu.SemaphoreType.DMA,
    ],
)


def _apply_body(values, scal, wd, out, tail32,
                buf0, buf1, obuf0, obuf1, tbuf0, tbuf1,
                scal_v, wd_v,
                sem0, sem1, osem0, osem1, tsem0, tsem1):
    wid = _wid()
    base = wid * PER_W
    pltpu.sync_copy(scal, scal_v)
    pltpu.sync_copy(wd, wd_v)
    _start_in(values, base, 0, buf0, sem0)
    _start_in(values, base, 1, buf1, sem1)

    isd = _splat(scal_v, 2)
    nmi = _splat(scal_v, 4)
    p = _splat(scal_v, 5)
    q = _splat(scal_v, 6)

    def chunk_pair(i, carry):
        for sub, buf, sem, obuf, osem, tbuf, tsem in (
                (0, buf0, sem0, obuf0, osem0, tbuf0, tsem0),
                (1, buf1, sem1, obuf1, osem1, tbuf1, tsem1)):
            ci = 2 * i + sub
            _wait_in(values, buf, sem)

            @pl.when(ci >= 2)
            def _():
                pltpu.make_async_copy(obuf, out.at[pl.ds(0, CHUNK)],
                                      osem).wait()
                pltpu.make_async_copy(tbuf, tail32.at[pl.ds(0, CHUNK)],
                                      tsem).wait()

            ione, izero = carry

            @plsc.parallel_loop(0, CHUNK // L, 1, unroll=8)
            def _(j):
                v = buf[pl.ds(j * L, L)]
                t = v * p + q
                b = jnp.clip(t.astype(jnp.int32), 0, K - 1)
                w = plsc.load_gather(wd_v, [b])
                z = v * isd + nmi
                obuf[pl.ds(j * L, L)] = w * z
                tbuf[pl.ds(j * L, L)] = jnp.where(
                    (t < 1.0) | (t >= float(K - 1)), ione, izero)

            pltpu.make_async_copy(obuf, out.at[pl.ds(base + ci * CHUNK,
                                                     CHUNK)], osem).start()
            pltpu.make_async_copy(
                tbuf, tail32.at[pl.ds(base + ci * CHUNK, CHUNK)],
                tsem).start()

            @pl.when(ci + 2 < NCH)
            def _():
                _start_in(values, base, ci + 2, buf, sem)
        return carry

    lax.fori_loop(0, NCH // 2, chunk_pair,
                  (jnp.ones((L,), jnp.int32), jnp.zeros((L,), jnp.int32)))
    for obuf, osem, tbuf, tsem in ((obuf0, osem0, tbuf0, tsem0),
                                   (obuf1, osem1, tbuf1, tsem1)):
        pltpu.make_async_copy(obuf, out.at[pl.ds(0, CHUNK)], osem).wait()
        pltpu.make_async_copy(tbuf, tail32.at[pl.ds(0, CHUNK)],
                              tsem).wait()


_apply = pl.kernel(
    _apply_body,
    out_type=(
        jax.ShapeDtypeStruct((N,), jnp.float32),
        jax.ShapeDtypeStruct((N,), jnp.int32),
    ),
    mesh=_MESH,
    compiler_params=pltpu.CompilerParams(needs_layout_passes=False),
    scratch_types=[
        pltpu.VMEM((CHUNK,), jnp.float32),
        pltpu.VMEM((CHUNK,), jnp.float32),
        pltpu.VMEM((CHUNK,), jnp.float32),
        pltpu.VMEM((CHUNK,), jnp.float32),
        pltpu.VMEM((CHUNK,), jnp.int32),
        pltpu.VMEM((CHUNK,), jnp.int32),
        pltpu.VMEM((L,), jnp.float32),
        pltpu.VMEM((L,), jnp.float32),
        pltpu.SemaphoreType.DMA,
        pltpu.SemaphoreType.DMA,
        pltpu.SemaphoreType.DMA,
        pltpu.SemaphoreType.DMA,
        pltpu.SemaphoreType.DMA,
        pltpu.SemaphoreType.DMA,
    ],
)


def kernel(values, k):
    parts = _stats(values)
    n = jnp.float32(N)
    s = jnp.sum(parts[:, 0, :])
    sq = jnp.sum(parts[:, 1, :])
    vmax = jnp.max(parts[:, 2, :])
    vmin = jnp.min(parts[:, 3, :])
    mu = s / n
    var = sq / n - mu * mu
    sd = jnp.sqrt(jnp.clip(var, EPS))
    zmax = jnp.clip(jnp.maximum(jnp.abs(vmax - mu), jnp.abs(vmin - mu)) / sd,
                    3.0, 8.0)
    inv_h = (K / 2) / zmax
    scal = jnp.zeros((L,), jnp.float32)
    isd = 1.0 / sd
    scal = scal.at[1].set(mu).at[2].set(isd).at[3].set(zmax)
    scal = scal.at[4].set(-mu * isd)
    scal = scal.at[5].set(isd * inv_h)
    scal = scal.at[6].set((zmax - mu * isd) * inv_h)


    hists = _hist(values, scal)
    c = jnp.sum(hists, axis=0)
    pos = c > 0
    c_mean = jnp.where(jnp.any(pos),
                       jnp.sum(jnp.where(pos, c, 0.0)) /
                       jnp.maximum(jnp.sum(pos.astype(jnp.float32)), 1.0),
                       jnp.float32(1.0))
    wd_bins = jnp.clip(c_mean / (c + EPS), 1.0, WMAX)
    out, tail32 = _apply(values, scal, wd_bins)
    tail = tail32 != 0
    return out, c, wd_bins, tail
